# baseline (speedup 1.0000x reference)
import jax
import jax.numpy as jnp
from jax import lax
from jax.experimental import pallas as pl
from jax.experimental.pallas import tpu as pltpu
from jax.experimental.pallas import tpu_sc as plsc

_IMAGE_TOKEN_INDEX = -200

_L, _B, _T, _D = 33, 2, 2048, 768
_P = 576
_N = 3 * _B
_LP = 48
_NS_C = 12
_RPW = _P // _NS_C


def _sc_body(hs_ref, w_ref, ids_ref, msk_ref, g_ref, out1_ref, out2_ref,
             ids_v, msk_v, idx_v, idx2_v, seg_v, w_v, g_v, out1_v, pub_v,
             base_v, shared, stage_v, sem, sem2, sem3, sem4):
    c = lax.axis_index("c")
    s = lax.axis_index("s")
    i16 = lax.iota(jnp.int32, 16)

    @pl.when(s < 3)
    def _compute():
        n = c * 3 + s
        ich = pltpu.async_copy(ids_ref.at[pl.ds(c * _T, _T)], ids_v, sem)
        mch = pltpu.async_copy(msk_ref.at[pl.ds(c * _T, _T)], msk_v, sem2)
        pltpu.sync_copy(g_ref.at[pl.ds(n * _LP, _LP)], g_v)
        mch.wait()

        def min_pass(pred_fn):
            def body(i, carry):
                pos = i16 + 16 * i
                v = msk_v[pl.ds(16 * i, 16)]
                return jnp.minimum(carry, jnp.where(pred_fn(v, pos), pos, _T))
            return jnp.min(lax.fori_loop(
                0, _T // 16, body, jnp.full((16,), _T, jnp.int32)))

        c0 = min_pass(lambda v, pos: v != 0)
        c1 = min_pass(lambda v, pos: (v != 0) & (pos > c0))
        c2 = min_pass(lambda v, pos: (v != 0) & (pos > c1))
        col = jnp.where(s == 0, c0, jnp.where(s == 1, c1, c2))

        for k in range(_LP // 16):
            lv = jnp.minimum(i16 + 16 * k, _L - 1)
            idx_v[pl.ds(16 * k, 16)] = lv * (_B * _T) + c * _T + col
        gch = pltpu.async_copy(hs_ref.at[idx_v], seg_v, sem2)
        wch = pltpu.async_copy(w_ref, w_v, sem3)

        ich.wait()

        def ibody(i, carry):
            v = ids_v[pl.ds(16 * i, 16)]
            return jnp.minimum(
                carry,
                jnp.where(v == _IMAGE_TOKEN_INDEX, i16 + 16 * i, _T))

        img = jnp.min(lax.fori_loop(
            0, _T // 16, ibody, jnp.full((16,), _T, jnp.int32)))
        gch.wait()
        wch.wait()

        def lbody(l, carry):
            lg0, lg1, lg2 = carry
            acc = seg_v[l, pl.ds(0, 16)] * w_v[l, pl.ds(0, 16)]
            for k in range(1, _D // 16):
                acc = acc + seg_v[l, pl.ds(16 * k, 16)] * w_v[l, pl.ds(16 * k, 16)]
            tot = jnp.sum(acc)
            return (jnp.where(i16 == l, tot, lg0),
                    jnp.where(i16 + 16 == l, tot, lg1),
                    jnp.where(i16 + 32 == l, tot, lg2))

        ninf = jnp.full((16,), -jnp.inf, jnp.float32)
        lg0, lg1, lg2 = lax.fori_loop(0, _L, lbody, (ninf, ninf, ninf))

        v0 = lg0 + g_v[pl.ds(0, 16)]
        v1 = lg1 + g_v[pl.ds(16, 16)]
        v2 = lg2 + g_v[pl.ds(32, 16)]
        mx = jnp.maximum(jnp.maximum(jnp.max(v0), jnp.max(v1)), jnp.max(v2))
        big = jnp.int32(1 << 20)
        barg = jnp.minimum(
            jnp.minimum(jnp.min(jnp.where(v0 >= mx, i16, big)),
                        jnp.min(jnp.where(v1 >= mx, i16 + 16, big))),
            jnp.min(jnp.where(v2 >= mx, i16 + 32, big)))

        for k in range(_D // 16):
            out1_v[pl.ds(16 * k, 16)] = seg_v[barg, pl.ds(16 * k, 16)]
        pltpu.sync_copy(out1_v, out1_ref.at[pl.ds(n * _D, _D)])
        rb = barg * (_B * _T) + c * _T + img
        pub_v[...] = jnp.zeros((16,), jnp.int32) + rb
        pltpu.sync_copy(pub_v, shared.at[pl.ds(16 * s, 16)])

    plsc.subcore_barrier()

    @pl.when(s < _NS_C)
    def _spans():
        pltpu.sync_copy(shared, base_v)
        bufs = (stage_v, seg_v.at[pl.ds(0, _RPW)])
        idxb = (idx_v, idx2_v)
        gsem = (sem, sem2)
        wsem = (sem3, sem4)

        def fill_idx(t, b):
            rb = base_v[pl.ds(16 * t, 16)][0]
            for k in range(_RPW // 16):
                idxb[b][pl.ds(16 * k, 16)] = rb + s * _RPW + i16 + 16 * k

        def start_write(t, b):
            dst0 = (c * 3 + t) * _P + s * _RPW
            return pltpu.async_copy(
                bufs[b], out2_ref.at[pl.ds(dst0, _RPW)], wsem[b])

        fill_idx(0, 0)
        gh = [pltpu.async_copy(hs_ref.at[idxb[0]], bufs[0], gsem[0]), None]
        wh = [None, None]
        for t in range(3):
            b = t % 2
            gh[b].wait()
            wh[b] = start_write(t, b)
            if t + 1 < 3:
                nb = 1 - b
                if t >= 1:
                    wh[nb].wait()
                fill_idx(t + 1, nb)
                gh[nb] = pltpu.async_copy(
                    hs_ref.at[idxb[nb]], bufs[nb], gsem[nb])
        wh[0].wait()
        wh[1].wait()


def kernel(hidden_states, W, input_ids, seg_token_mask, num_patches):
    del num_patches
    L, B, T, D = hidden_states.shape
    hs_flat = hidden_states.reshape(L * B * T, D)
    ids1 = input_ids.reshape(B * T)
    msk1 = seg_token_mask.astype(jnp.int32).reshape(B * T)

    g = jax.random.gumbel(jax.random.key(1), (_N, _L), jnp.float32)
    g_pad = jnp.concatenate(
        [g, jnp.zeros((_N, _LP - _L), jnp.float32)], axis=1).reshape(_N * _LP)

    mesh = plsc.VectorSubcoreMesh(core_axis_name="c", subcore_axis_name="s")
    out1f, out2f = pl.kernel(
        _sc_body,
        out_type=(
            jax.ShapeDtypeStruct((_N * _D,), jnp.float32),
            jax.ShapeDtypeStruct((_N * _P, _D), jnp.float32),
        ),
        mesh=mesh,
        compiler_params=pltpu.CompilerParams(needs_layout_passes=False),
        scratch_types=[
            pltpu.VMEM((_T,), jnp.int32),
            pltpu.VMEM((_T,), jnp.int32),
            pltpu.VMEM((_LP,), jnp.int32),
            pltpu.VMEM((_LP,), jnp.int32),
            pltpu.VMEM((_LP, _D), jnp.float32),
            pltpu.VMEM((_L, _D), jnp.float32),
            pltpu.VMEM((_LP,), jnp.float32),
            pltpu.VMEM((_D,), jnp.float32),
            pltpu.VMEM((16,), jnp.int32),
            pltpu.VMEM((_LP,), jnp.int32),
            pltpu.VMEM_SHARED((_LP,), jnp.int32),
            pltpu.VMEM((_RPW, _D), jnp.float32),
            pltpu.SemaphoreType.DMA,
            pltpu.SemaphoreType.DMA,
            pltpu.SemaphoreType.DMA,
            pltpu.SemaphoreType.DMA,
        ],
    )(hs_flat, W, ids1, msk1, g_pad)

    out1 = out1f.reshape(_N, _D)
    out2 = out2f.reshape(_N, _P, _D)
    return (out1, out2, out1)


# speedup vs baseline: 1.0568x; 1.0568x over previous
import jax
import jax.numpy as jnp
from jax import lax
from jax.experimental import pallas as pl
from jax.experimental.pallas import tpu as pltpu
from jax.experimental.pallas import tpu_sc as plsc

_IMAGE_TOKEN_INDEX = -200

_L, _B, _T, _D = 33, 2, 2048, 768
_P = 576
_N = 3 * _B
_LP = 48
_NS_C = 12
_RPW = _P // _NS_C


def _sc_body(hs_ref, w_ref, meta_ref, msk_ref, g_ref, out1_ref, out2_ref,
             meta_v, msk_v, idx_v, idx2_v, seg_v, w_v, g_v, out1_v, pub_v,
             base_v, shared, stage_v, sem, sem2, sem3, sem4):
    c = lax.axis_index("c")
    s = lax.axis_index("s")
    i16 = lax.iota(jnp.int32, 16)

    @pl.when(s < 3)
    def _compute():
        n = c * 3 + s
        tch = pltpu.async_copy(meta_ref, meta_v, sem)
        mch = pltpu.async_copy(msk_ref.at[pl.ds(c * _T, _T)], msk_v, sem2)
        gmch = pltpu.async_copy(g_ref.at[pl.ds(n * _LP, _LP)], g_v, sem3)
        wch = pltpu.async_copy(w_ref, w_v, sem4)
        mch.wait()

        def body(i, carry):
            m0, m1, m2 = carry
            v = msk_v[pl.ds(16 * i, 16)]
            p = jnp.where(v != 0, i16 + 16 * i, _T)
            t0 = jnp.minimum(m0, p)
            t1 = jnp.maximum(m0, p)
            u1 = jnp.minimum(m1, t1)
            u2 = jnp.maximum(m1, t1)
            v2 = jnp.minimum(m2, u2)
            return (t0, u1, v2)

        topT = jnp.full((16,), _T, jnp.int32)
        m0, m1, m2 = lax.fori_loop(0, _T // 16, body, (topT, topT, topT))
        c0 = jnp.min(m0)
        fmax = lambda m: jnp.max(jnp.where(m < _T, m, -1))
        c2 = jnp.maximum(jnp.maximum(fmax(m0), fmax(m1)), fmax(m2))
        fsum = lambda m: jnp.sum(jnp.where(m < _T, m, 0))
        c1 = (fsum(m0) + fsum(m1) + fsum(m2)) - c0 - c2
        col = jnp.where(s == 0, c0, jnp.where(s == 1, c1, c2))

        for k in range(_LP // 16):
            lv = jnp.minimum(i16 + 16 * k, _L - 1)
            idx_v[pl.ds(16 * k, 16)] = lv * (_B * _T) + c * _T + col
        gch = pltpu.async_copy(hs_ref.at[idx_v], seg_v, sem2)

        tch.wait()
        img = jnp.sum(jnp.where(i16 == c, meta_v[...], 0))
        gch.wait()
        gmch.wait()
        wch.wait()

        def lbody(l, carry):
            lg0, lg1, lg2 = carry
            acc = seg_v[l, pl.ds(0, 16)] * w_v[l, pl.ds(0, 16)]
            for k in range(1, _D // 16):
                acc = acc + seg_v[l, pl.ds(16 * k, 16)] * w_v[l, pl.ds(16 * k, 16)]
            tot = jnp.sum(acc)
            return (jnp.where(i16 == l, tot, lg0),
                    jnp.where(i16 + 16 == l, tot, lg1),
                    jnp.where(i16 + 32 == l, tot, lg2))

        ninf = jnp.full((16,), -jnp.inf, jnp.float32)
        lg0, lg1, lg2 = lax.fori_loop(0, _L, lbody, (ninf, ninf, ninf))

        v0 = lg0 + g_v[pl.ds(0, 16)]
        v1 = lg1 + g_v[pl.ds(16, 16)]
        v2 = lg2 + g_v[pl.ds(32, 16)]
        mx = jnp.maximum(jnp.maximum(jnp.max(v0), jnp.max(v1)), jnp.max(v2))
        big = jnp.int32(1 << 20)
        barg = jnp.minimum(
            jnp.minimum(jnp.min(jnp.where(v0 >= mx, i16, big)),
                        jnp.min(jnp.where(v1 >= mx, i16 + 16, big))),
            jnp.min(jnp.where(v2 >= mx, i16 + 32, big)))

        for k in range(_D // 16):
            out1_v[pl.ds(16 * k, 16)] = seg_v[barg, pl.ds(16 * k, 16)]
        pltpu.sync_copy(out1_v, out1_ref.at[pl.ds(n * _D, _D)])
        rb = barg * (_B * _T) + c * _T + img
        pub_v[...] = jnp.zeros((16,), jnp.int32) + rb
        pltpu.sync_copy(pub_v, shared.at[pl.ds(16 * s, 16)])

    plsc.subcore_barrier()

    @pl.when(s < _NS_C)
    def _spans():
        pltpu.sync_copy(shared, base_v)
        bufs = (stage_v, seg_v.at[pl.ds(0, _RPW)])
        idxb = (idx_v, idx2_v)
        gsem = (sem, sem2)
        wsem = (sem3, sem4)

        def fill_idx(t, b):
            rb = base_v[pl.ds(16 * t, 16)][0]
            for k in range(_RPW // 16):
                idxb[b][pl.ds(16 * k, 16)] = rb + s * _RPW + i16 + 16 * k

        def start_write(t, b):
            dst0 = (c * 3 + t) * _P + s * _RPW
            return pltpu.async_copy(
                bufs[b], out2_ref.at[pl.ds(dst0, _RPW)], wsem[b])

        fill_idx(0, 0)
        gh = [pltpu.async_copy(hs_ref.at[idxb[0]], bufs[0], gsem[0]), None]
        wh = [None, None]
        for t in range(3):
            b = t % 2
            gh[b].wait()
            wh[b] = start_write(t, b)
            if t + 1 < 3:
                nb = 1 - b
                if t >= 1:
                    wh[nb].wait()
                fill_idx(t + 1, nb)
                gh[nb] = pltpu.async_copy(
                    hs_ref.at[idxb[nb]], bufs[nb], gsem[nb])
        wh[0].wait()
        wh[1].wait()


def kernel(hidden_states, W, input_ids, seg_token_mask, num_patches):
    del num_patches
    L, B, T, D = hidden_states.shape
    hs_flat = hidden_states.reshape(L * B * T, D)
    msk1 = seg_token_mask.astype(jnp.int32).reshape(B * T)
    it = jnp.arange(T, dtype=jnp.int32)[None, :]
    img_idx = jnp.min(
        jnp.where(input_ids == _IMAGE_TOKEN_INDEX, it, T), axis=1)
    meta = jnp.concatenate(
        [img_idx.astype(jnp.int32), jnp.zeros((16 - _B,), jnp.int32)])

    g = jax.random.gumbel(jax.random.key(1), (_N, _L), jnp.float32)
    g_pad = jnp.concatenate(
        [g, jnp.zeros((_N, _LP - _L), jnp.float32)], axis=1).reshape(_N * _LP)

    mesh = plsc.VectorSubcoreMesh(core_axis_name="c", subcore_axis_name="s")
    out1f, out2f = pl.kernel(
        _sc_body,
        out_type=(
            jax.ShapeDtypeStruct((_N * _D,), jnp.float32),
            jax.ShapeDtypeStruct((_N * _P, _D), jnp.float32),
        ),
        mesh=mesh,
        compiler_params=pltpu.CompilerParams(needs_layout_passes=False),
        scratch_types=[
            pltpu.VMEM((16,), jnp.int32),
            pltpu.VMEM((_T,), jnp.int32),
            pltpu.VMEM((_LP,), jnp.int32),
            pltpu.VMEM((_LP,), jnp.int32),
            pltpu.VMEM((_LP, _D), jnp.float32),
            pltpu.VMEM((_L, _D), jnp.float32),
            pltpu.VMEM((_LP,), jnp.float32),
            pltpu.VMEM((_D,), jnp.float32),
            pltpu.VMEM((16,), jnp.int32),
            pltpu.VMEM((_LP,), jnp.int32),
            pltpu.VMEM_SHARED((_LP,), jnp.int32),
            pltpu.VMEM((_RPW, _D), jnp.float32),
            pltpu.SemaphoreType.DMA,
            pltpu.SemaphoreType.DMA,
            pltpu.SemaphoreType.DMA,
            pltpu.SemaphoreType.DMA,
        ],
    )(hs_flat, W, meta, msk1, g_pad)

    out1 = out1f.reshape(_N, _D)
    out2 = out2f.reshape(_N, _P, _D)
    return (out1, out2, out1)
